# baseline (speedup 1.0000x reference)
import functools

import jax
import jax.numpy as jnp
from jax import lax
from jax.experimental import pallas as pl
from jax.experimental.pallas import tpu as pltpu
from jax.experimental.pallas import tpu_sc as plsc

B = 16384
D = 64
G = 8
NUM_GENRES = 32

NC = 2
NS = 16
NW = NC * NS
BPW = B // NW
CHUNK = 128
KCH = BPW // CHUNK

BLK = 1024


def _sc_gather(table2, pidx3, idxf):
    mesh = plsc.VectorSubcoreMesh(
        core_axis_name="c", subcore_axis_name="s",
        num_cores=NC, num_subcores=NS)

    @functools.partial(
        pl.kernel,
        out_type=jax.ShapeDtypeStruct((B, D), jnp.float32),
        mesh=mesh,
        scratch_types=[
            pltpu.VMEM((KCH, CHUNK), jnp.int32),
            pltpu.VMEM((BPW,), jnp.int32),
            pltpu.VMEM((BPW, 2 * D), jnp.float32),
            pltpu.VMEM((BPW, D), jnp.float32),
            pltpu.SemaphoreType.DMA,
        ],
        compiler_params=pltpu.CompilerParams(
            use_tc_tiling_on_sc=False, needs_layout_passes=False),
    )
    def k(table_hbm, pidx_hbm, idxf_hbm, out_hbm,
          pidx_v, idxf_v, rows_v, stage_v, sem):
        wid = lax.axis_index("s") * NC + lax.axis_index("c")
        pltpu.sync_copy(pidx_hbm.at[wid], pidx_v)
        pltpu.sync_copy(idxf_hbm.at[wid], idxf_v)
        cps = []
        for j in range(KCH):
            cps.append(pltpu.async_copy(
                table_hbm.at[pidx_v.at[j]],
                rows_v.at[pl.ds(j * CHUNK, CHUNK)],
                sem))
        for cp in cps:
            cp.wait()

        lanes = lax.iota(jnp.int32, 16)

        def select16(kk, _):
            row16 = kk * 16 + lanes
            par = idxf_v[pl.ds(kk * 16, 16)] & 1
            col_base = par * D
            for d in range(D):
                x = plsc.load_gather(rows_v, [row16, col_base + d])
                plsc.store_scatter(stage_v, [row16, lanes * 0 + d], x)
            return 0

        lax.fori_loop(0, BPW // 16, select16, 0)

        pltpu.sync_copy(stage_v, out_hbm.at[pl.ds(wid * BPW, BPW)])

    return k(table2, pidx3, idxf)


def _tc_body(title_ref, genres_ref, gt_ref, w_ref, b_ref, out_ref):
    g = genres_ref[...]
    cls = lax.broadcasted_iota(jnp.int32, (1, NUM_GENRES), 1)
    counts = jnp.zeros((BLK, NUM_GENRES), jnp.float32)
    for j in range(G):
        counts += (g[:, j:j + 1] == cls).astype(jnp.float32)
    genre_emb = jnp.dot(counts, gt_ref[...],
                        preferred_element_type=jnp.float32)
    comb = jnp.concatenate([title_ref[...], genre_emb], axis=1)
    out = jnp.dot(comb, w_ref[...],
                  preferred_element_type=jnp.float32) + b_ref[...]
    out_ref[...] = jnp.maximum(out, 0.0)


def _tc_combine(title_g, movie_genres, genre_table, W, b2):
    return pl.pallas_call(
        _tc_body,
        out_shape=jax.ShapeDtypeStruct((B, D), jnp.float32),
        grid=(B // BLK,),
        in_specs=[
            pl.BlockSpec((BLK, D), lambda i: (i, 0)),
            pl.BlockSpec((BLK, G), lambda i: (i, 0)),
            pl.BlockSpec((NUM_GENRES, D), lambda i: (0, 0)),
            pl.BlockSpec((2 * D, D), lambda i: (0, 0)),
            pl.BlockSpec((1, D), lambda i: (0, 0)),
        ],
        out_specs=pl.BlockSpec((BLK, D), lambda i: (i, 0)),
    )(title_g, movie_genres, genre_table, W, b2)


def kernel(movie_title, movie_genres, title_table, genre_table, W, b):
    table2 = title_table.reshape(500000, 2 * D)
    pidx3 = (movie_title >> 1).reshape(NW, KCH, CHUNK)
    idxf = movie_title.reshape(NW, BPW)
    title_g = _sc_gather(table2, pidx3, idxf)
    return _tc_combine(title_g, movie_genres, genre_table, W,
                       b.reshape(1, D))
